# baseline (speedup 1.0000x reference)
import jax
import jax.numpy as jnp
from jax import lax
from jax.experimental import pallas as pl
from jax.experimental.pallas import tpu as pltpu
from jax.experimental.pallas import tpu_sc as plsc

N = 10000
D = 256
H = 128
NPAD = 10240
NC = 2
NS = 16
CHUNK = 128
STRIPE = NPAD // NS

_mesh = plsc.VectorSubcoreMesh(core_axis_name="c", subcore_axis_name="s")


def _deg_body(dst_hbm, ones_hbm, zeros_hbm, deg_hbm,
              dstv, onesv, deg_spmem, sem):
    c = lax.axis_index("c")
    s = lax.axis_index("s")
    wid = c * NS + s
    pltpu.sync_copy(dst_hbm.at[wid], dstv)
    pltpu.sync_copy(ones_hbm, onesv)
    pltpu.sync_copy(zeros_hbm, deg_spmem.at[pl.ds(s * STRIPE, STRIPE)])
    plsc.subcore_barrier()

    n_chunks = dstv.shape[0]

    def chunk(g, carry):
        pltpu.sync_copy(onesv, deg_spmem.at[dstv.at[g]], add=True)
        return carry

    lax.fori_loop(0, n_chunks, chunk, 0)
    plsc.subcore_barrier()
    pltpu.sync_copy(deg_spmem.at[pl.ds(s * STRIPE, STRIPE)],
                    deg_hbm.at[c, pl.ds(s * STRIPE, STRIPE)])


def _make_deg_kernel(n_chunks):
    return pl.kernel(
        _deg_body,
        out_type=jax.ShapeDtypeStruct((NC, NPAD, H), jnp.float32),
        mesh=_mesh,
        scratch_types=[
            pltpu.VMEM((n_chunks, CHUNK), jnp.int32),
            pltpu.VMEM((CHUNK, H), jnp.float32),
            pltpu.VMEM_SHARED((NPAD, H), jnp.float32),
            pltpu.SemaphoreType.DMA,
        ],
    )


def _agg_body(src_hbm, dst_hbm, h20_hbm, h21_hbm, zeros_hbm,
              agg0_hbm, agg1_hbm, srcv, dstv, rows0, rows1, agg_spmem,
              sem0, sem1):
    c = lax.axis_index("c")
    s = lax.axis_index("s")
    pltpu.sync_copy(zeros_hbm, agg_spmem.at[pl.ds(s * STRIPE, STRIPE)])
    plsc.subcore_barrier()

    per_phase = srcv.shape[0]
    n_phases = src_hbm.shape[1] // per_phase
    n_pairs = per_phase // 2

    def run(table_hbm, out_hbm):
        for ph in range(n_phases):
            pltpu.sync_copy(src_hbm.at[s, pl.ds(ph * per_phase, per_phase)],
                            srcv)
            pltpu.sync_copy(dst_hbm.at[s, pl.ds(ph * per_phase, per_phase)],
                            dstv)
            pltpu.async_copy(table_hbm.at[srcv.at[0]], rows0, sem0)

            def pair(p, carry):
                g0 = p * 2
                g1 = g0 + 1
                pltpu.async_copy(table_hbm.at[srcv.at[g1]], rows1, sem1)
                pltpu.make_async_copy(table_hbm.at[srcv.at[g0]], rows0,
                                      sem0).wait()
                pltpu.sync_copy(rows0, agg_spmem.at[dstv.at[g0]], add=True)

                @pl.when(g0 + 2 < per_phase)
                def _():
                    pltpu.async_copy(table_hbm.at[srcv.at[g0 + 2]], rows0,
                                     sem0)

                pltpu.make_async_copy(table_hbm.at[srcv.at[g1]], rows1,
                                      sem1).wait()
                pltpu.sync_copy(rows1, agg_spmem.at[dstv.at[g1]], add=True)
                return carry

            lax.fori_loop(0, n_pairs, pair, 0)

        plsc.subcore_barrier()
        pltpu.sync_copy(agg_spmem.at[pl.ds(s * STRIPE, STRIPE)],
                        out_hbm.at[pl.ds(s * STRIPE, STRIPE)])

    @pl.when(c == 0)
    def _():
        run(h20_hbm, agg0_hbm)

    @pl.when(c == 1)
    def _():
        run(h21_hbm, agg1_hbm)


def _make_agg_kernel(n_chunks):
    per_phase = n_chunks // 2
    assert per_phase % 2 == 0 and n_chunks % per_phase == 0
    return pl.kernel(
        _agg_body,
        out_type=[jax.ShapeDtypeStruct((NPAD, H), jnp.float32),
                  jax.ShapeDtypeStruct((NPAD, H), jnp.float32)],
        mesh=_mesh,
        scratch_types=[
            pltpu.VMEM((per_phase, CHUNK), jnp.int32),
            pltpu.VMEM((per_phase, CHUNK), jnp.int32),
            pltpu.VMEM((CHUNK, H), jnp.float32),
            pltpu.VMEM((CHUNK, H), jnp.float32),
            pltpu.VMEM_SHARED((NPAD, H), jnp.float32),
            pltpu.SemaphoreType.DMA,
            pltpu.SemaphoreType.DMA,
        ],
    )


_BLK = 1000


def _mm_body(x_ref, w_ref, h_ref):
    h_ref[...] = jnp.dot(x_ref[...], w_ref[...],
                         preferred_element_type=jnp.float32)


def _mm_call(x, W):
    grid = (N // _BLK,)
    return pl.pallas_call(
        _mm_body,
        grid=grid,
        in_specs=[
            pl.BlockSpec((_BLK, D), lambda i: (i, 0)),
            pl.BlockSpec((D, D), lambda i: (0, 0)),
        ],
        out_specs=pl.BlockSpec((_BLK, D), lambda i: (i, 0)),
        out_shape=jax.ShapeDtypeStruct((N, D), jnp.float32),
    )(x, W)


def _h2_body(h_ref, deg_ref, h20_ref, h21_ref):
    d = deg_ref[0, :, 0:1] + deg_ref[1, :, 0:1] + 1.0
    dinv = lax.rsqrt(d)
    h2 = h_ref[...] * dinv
    h20_ref[...] = h2[:, :H]
    h21_ref[...] = h2[:, H:]


def _h2_call(h, deg):
    grid = (N // _BLK,)
    return pl.pallas_call(
        _h2_body,
        grid=grid,
        in_specs=[
            pl.BlockSpec((_BLK, D), lambda i: (i, 0)),
            pl.BlockSpec((NC, _BLK, H), lambda i: (0, i, 0)),
        ],
        out_specs=[
            pl.BlockSpec((_BLK, H), lambda i: (i, 0)),
            pl.BlockSpec((_BLK, H), lambda i: (i, 0)),
        ],
        out_shape=[jax.ShapeDtypeStruct((N, H), jnp.float32),
                   jax.ShapeDtypeStruct((N, H), jnp.float32)],
    )(h, deg)


def _final_body(x_ref, agg0_ref, agg1_ref, h20_ref, h21_ref,
                deg_ref, b_ref, out_ref):
    d = deg_ref[0, :, 0:1] + deg_ref[1, :, 0:1] + 1.0
    dinv = lax.rsqrt(d)
    t0 = (agg0_ref[...] + h20_ref[...]) * dinv + b_ref[:, :H]
    t1 = (agg1_ref[...] + h21_ref[...]) * dinv + b_ref[:, H:]
    out_ref[:, :H] = x_ref[:, :H] + jnp.maximum(t0, 0.0)
    out_ref[:, H:] = x_ref[:, H:] + jnp.maximum(t1, 0.0)


def _final_call(x, agg0, agg1, h20, h21, deg, b2):
    grid = (N // _BLK,)
    return pl.pallas_call(
        _final_body,
        grid=grid,
        in_specs=[
            pl.BlockSpec((_BLK, D), lambda i: (i, 0)),
            pl.BlockSpec((_BLK, H), lambda i: (i, 0)),
            pl.BlockSpec((_BLK, H), lambda i: (i, 0)),
            pl.BlockSpec((_BLK, H), lambda i: (i, 0)),
            pl.BlockSpec((_BLK, H), lambda i: (i, 0)),
            pl.BlockSpec((NC, _BLK, H), lambda i: (0, i, 0)),
            pl.BlockSpec((1, D), lambda i: (0, 0)),
        ],
        out_specs=pl.BlockSpec((_BLK, D), lambda i: (i, 0)),
        out_shape=jax.ShapeDtypeStruct((N, D), jnp.float32),
    )(x, agg0, agg1, h20, h21, deg, b2)


def kernel(x, edge_index, W, b):
    E = edge_index.shape[1]
    epad = -(-E // (NC * NS * CHUNK)) * (NC * NS * CHUNK)
    npad_e = epad - E

    src = edge_index[0].astype(jnp.int32)
    dst = edge_index[1].astype(jnp.int32)
    pad_ids = jnp.arange(npad_e, dtype=jnp.int32)
    src_p = jnp.concatenate([src, pad_ids % N])
    dst_p = jnp.concatenate([dst, N + pad_ids % (NPAD - N)])

    dst_a = dst_p.reshape(NC * NS, -1, CHUNK)
    src_c = src_p.reshape(NS, -1, CHUNK)
    dst_c = dst_p.reshape(NS, -1, CHUNK)

    onesH = jnp.ones((CHUNK, H), jnp.float32)
    zerosH = jnp.zeros((STRIPE, H), jnp.float32)

    hmm = _mm_call(x, W)
    deg = _make_deg_kernel(dst_a.shape[1])(dst_a, onesH, zerosH)
    h20, h21 = _h2_call(hmm, deg)
    agg0, agg1 = _make_agg_kernel(src_c.shape[1])(src_c, dst_c, h20, h21, zerosH)
    out = _final_call(x, agg0, agg1, h20, h21, deg, b.reshape(1, D))
    return out


# speedup vs baseline: 1.0087x; 1.0087x over previous
import jax
import jax.numpy as jnp
from jax import lax
from jax.experimental import pallas as pl
from jax.experimental.pallas import tpu as pltpu
from jax.experimental.pallas import tpu_sc as plsc

N = 10000
D = 256
H = 128
NPAD = 10240
NC = 2
NS = 16
CHUNK = 128
STRIPE = NPAD // NS

_mesh = plsc.VectorSubcoreMesh(core_axis_name="c", subcore_axis_name="s")


def _deg_body(dst_hbm, ones_hbm, zeros_hbm, deg_hbm,
              dstv, onesv, deg_spmem, sem):
    c = lax.axis_index("c")
    s = lax.axis_index("s")
    wid = c * NS + s
    pltpu.sync_copy(dst_hbm.at[wid], dstv)
    pltpu.sync_copy(ones_hbm, onesv)
    pltpu.sync_copy(zeros_hbm, deg_spmem.at[pl.ds(s * STRIPE, STRIPE)])
    plsc.subcore_barrier()

    n_chunks = dstv.shape[0]

    def chunk(g, carry):
        pltpu.sync_copy(onesv, deg_spmem.at[dstv.at[g]], add=True)
        return carry

    lax.fori_loop(0, n_chunks, chunk, 0)
    plsc.subcore_barrier()
    pltpu.sync_copy(deg_spmem.at[pl.ds(s * STRIPE, STRIPE)],
                    deg_hbm.at[c, pl.ds(s * STRIPE, STRIPE)])


def _make_deg_kernel(n_chunks):
    return pl.kernel(
        _deg_body,
        out_type=jax.ShapeDtypeStruct((NC, NPAD, H), jnp.float32),
        mesh=_mesh,
        scratch_types=[
            pltpu.VMEM((n_chunks, CHUNK), jnp.int32),
            pltpu.VMEM((CHUNK, H), jnp.float32),
            pltpu.VMEM_SHARED((NPAD, H), jnp.float32),
            pltpu.SemaphoreType.DMA,
        ],
    )


def _agg_body(src_hbm, dst_hbm, h20_hbm, h21_hbm, zeros_hbm,
              agg0_hbm, agg1_hbm, srcv, dstv, rows0, rows1, agg_spmem,
              sem0, sem1):
    c = lax.axis_index("c")
    s = lax.axis_index("s")
    pltpu.sync_copy(zeros_hbm, agg_spmem.at[pl.ds(s * STRIPE, STRIPE)])
    plsc.subcore_barrier()

    per_phase = srcv.shape[0]
    n_phases = src_hbm.shape[1] // per_phase
    n_pairs = per_phase // 2

    def run(table_hbm, out_hbm):
        for ph in range(n_phases):
            pltpu.sync_copy(src_hbm.at[s, pl.ds(ph * per_phase, per_phase)],
                            srcv)
            pltpu.sync_copy(dst_hbm.at[s, pl.ds(ph * per_phase, per_phase)],
                            dstv)
            pltpu.async_copy(table_hbm.at[srcv.at[0]], rows0, sem0)

            def pair(p, carry):
                g0 = p * 2
                g1 = g0 + 1
                pltpu.async_copy(table_hbm.at[srcv.at[g1]], rows1, sem1)
                pltpu.make_async_copy(table_hbm.at[srcv.at[g0]], rows0,
                                      sem0).wait()
                pltpu.sync_copy(rows0, agg_spmem.at[dstv.at[g0]], add=True)

                @pl.when(g0 + 2 < per_phase)
                def _():
                    pltpu.async_copy(table_hbm.at[srcv.at[g0 + 2]], rows0,
                                     sem0)

                pltpu.make_async_copy(table_hbm.at[srcv.at[g1]], rows1,
                                      sem1).wait()
                pltpu.sync_copy(rows1, agg_spmem.at[dstv.at[g1]], add=True)
                return carry

            lax.fori_loop(0, n_pairs, pair, 0)

        plsc.subcore_barrier()
        pltpu.sync_copy(agg_spmem.at[pl.ds(s * STRIPE, STRIPE)],
                        out_hbm.at[pl.ds(s * STRIPE, STRIPE)])

    @pl.when(c == 0)
    def _():
        run(h20_hbm, agg0_hbm)

    @pl.when(c == 1)
    def _():
        run(h21_hbm, agg1_hbm)


def _make_agg_kernel(n_chunks):
    per_phase = n_chunks // 2
    assert per_phase % 2 == 0 and n_chunks % per_phase == 0
    return pl.kernel(
        _agg_body,
        out_type=[jax.ShapeDtypeStruct((NPAD, H), jnp.float32),
                  jax.ShapeDtypeStruct((NPAD, H), jnp.float32)],
        mesh=_mesh,
        scratch_types=[
            pltpu.VMEM((per_phase, CHUNK), jnp.int32),
            pltpu.VMEM((per_phase, CHUNK), jnp.int32),
            pltpu.VMEM((CHUNK, H), jnp.float32),
            pltpu.VMEM((CHUNK, H), jnp.float32),
            pltpu.VMEM_SHARED((NPAD, H), jnp.float32),
            pltpu.SemaphoreType.DMA,
            pltpu.SemaphoreType.DMA,
        ],
    )


_BLK = 1000


def _h2_body(x_ref, w_ref, deg_ref, h20_ref, h21_ref):
    d = deg_ref[0, :, 0:1] + deg_ref[1, :, 0:1] + 1.0
    dinv = lax.rsqrt(d)
    h = jnp.dot(x_ref[...], w_ref[...], preferred_element_type=jnp.float32)
    h2 = h * dinv
    h20_ref[...] = h2[:, :H]
    h21_ref[...] = h2[:, H:]


def _h2_call(x, W, deg):
    grid = (N // _BLK,)
    return pl.pallas_call(
        _h2_body,
        grid=grid,
        in_specs=[
            pl.BlockSpec((_BLK, D), lambda i: (i, 0)),
            pl.BlockSpec((D, D), lambda i: (0, 0)),
            pl.BlockSpec((NC, _BLK, H), lambda i: (0, i, 0)),
        ],
        out_specs=[
            pl.BlockSpec((_BLK, H), lambda i: (i, 0)),
            pl.BlockSpec((_BLK, H), lambda i: (i, 0)),
        ],
        out_shape=[jax.ShapeDtypeStruct((N, H), jnp.float32),
                   jax.ShapeDtypeStruct((N, H), jnp.float32)],
    )(x, W, deg)


def _final_body(x_ref, agg0_ref, agg1_ref, h20_ref, h21_ref,
                deg_ref, b_ref, out_ref):
    d = deg_ref[0, :, 0:1] + deg_ref[1, :, 0:1] + 1.0
    dinv = lax.rsqrt(d)
    t0 = (agg0_ref[...] + h20_ref[...]) * dinv + b_ref[:, :H]
    t1 = (agg1_ref[...] + h21_ref[...]) * dinv + b_ref[:, H:]
    out_ref[:, :H] = x_ref[:, :H] + jnp.maximum(t0, 0.0)
    out_ref[:, H:] = x_ref[:, H:] + jnp.maximum(t1, 0.0)


def _final_call(x, agg0, agg1, h20, h21, deg, b2):
    grid = (N // _BLK,)
    return pl.pallas_call(
        _final_body,
        grid=grid,
        in_specs=[
            pl.BlockSpec((_BLK, D), lambda i: (i, 0)),
            pl.BlockSpec((_BLK, H), lambda i: (i, 0)),
            pl.BlockSpec((_BLK, H), lambda i: (i, 0)),
            pl.BlockSpec((_BLK, H), lambda i: (i, 0)),
            pl.BlockSpec((_BLK, H), lambda i: (i, 0)),
            pl.BlockSpec((NC, _BLK, H), lambda i: (0, i, 0)),
            pl.BlockSpec((1, D), lambda i: (0, 0)),
        ],
        out_specs=pl.BlockSpec((_BLK, D), lambda i: (i, 0)),
        out_shape=jax.ShapeDtypeStruct((N, D), jnp.float32),
    )(x, agg0, agg1, h20, h21, deg, b2)


def kernel(x, edge_index, W, b):
    E = edge_index.shape[1]
    epad = -(-E // (NC * NS * CHUNK)) * (NC * NS * CHUNK)
    npad_e = epad - E

    src = edge_index[0].astype(jnp.int32)
    dst = edge_index[1].astype(jnp.int32)
    pad_ids = jnp.arange(npad_e, dtype=jnp.int32)
    src_p = jnp.concatenate([src, pad_ids % N])
    dst_p = jnp.concatenate([dst, N + pad_ids % (NPAD - N)])

    dst_a = dst_p.reshape(NC * NS, -1, CHUNK)
    src_c = src_p.reshape(NS, -1, CHUNK)
    dst_c = dst_p.reshape(NS, -1, CHUNK)

    onesH = jnp.ones((CHUNK, H), jnp.float32)
    zerosH = jnp.zeros((STRIPE, H), jnp.float32)

    deg = _make_deg_kernel(dst_a.shape[1])(dst_a, onesH, zerosH)
    h20, h21 = _h2_call(x, W, deg)
    agg0, agg1 = _make_agg_kernel(src_c.shape[1])(src_c, dst_c, h20, h21, zerosH)
    out = _final_call(x, agg0, agg1, h20, h21, deg, b.reshape(1, D))
    return out
